# baseline (speedup 1.0000x reference)
import functools

import jax
import jax.numpy as jnp
import numpy as np
from jax import lax
from jax.experimental import pallas as pl
from jax.experimental.pallas import tpu as pltpu
from jax.experimental.pallas import tpu_sc as plsc

N = 10000
E = 320000
D = 128
HID = 128
LAT = 64
COND = 5

NCORES = 2
NSUB = 16
HALF = D // 2
CHUNK = 128
NROWS = NSUB * 160
CPT = NROWS // NSUB
NSTAGE = 2
CPS = CPT // NSTAGE
NBUF = 8
E_PAD = NROWS * CHUNK
N_PAD = 10112
RPT = N_PAD // NSUB
HPT = N // NSUB
HPIECE = 125

BN = 400
GRID = N // BN

_PAD_I = np.arange(E_PAD - E, dtype=np.int32)
_PAD_SRC = _PAD_I % N
_PAD_DST = (N + _PAD_I % (N_PAD - N)).astype(np.int32)


def _make_two_convs():
    mesh = plsc.VectorSubcoreMesh(core_axis_name="c", subcore_axis_name="s",
                                  num_cores=NCORES, num_subcores=NSUB)

    @functools.partial(
        pl.kernel,
        out_type=(jax.ShapeDtypeStruct((NCORES * N, HALF), jnp.float32),
                  jax.ShapeDtypeStruct((NCORES, N_PAD, HALF), jnp.float32)),
        mesh=mesh,
        compiler_params=pltpu.CompilerParams(use_tc_tiling_on_sc=False),
        scratch_types=(
            [pltpu.VMEM((CPS, CHUNK), jnp.int32),
             pltpu.VMEM((CPS, CHUNK), jnp.int32),
             pltpu.VMEM((64,), jnp.float32)]
            + [pltpu.VMEM((CHUNK, HALF), jnp.float32)] * NBUF
            + [pltpu.VMEM_SHARED((N_PAD, HALF), jnp.float32)]
            + [pltpu.SemaphoreType.DMA] * (2 * NBUF)
        ),
    )
    def two_convs(g1_hbm, b1_hbm, src_hbm, dst_hbm, zeros_hbm,
                  h1_hbm, out_hbm, src_v, dst_v, b1v, *rest):
        rows = list(rest[:NBUF])
        acc = rest[NBUF]
        sg = list(rest[NBUF + 1:NBUF + 1 + NBUF])
        ss = list(rest[NBUF + 1 + NBUF:])
        c = lax.axis_index("c")
        s = lax.axis_index("s")

        def zero_acc():
            pltpu.sync_copy(zeros_hbm.at[pl.ds(s * RPT, RPT)],
                            acc.at[pl.ds(s * RPT, RPT)])

        def segsum(table_hbm):
            def stage(st, carry):
                base = s * CPT + st * CPS
                pltpu.sync_copy(src_hbm.at[pl.ds(c * NROWS + base, CPS)],
                                src_v)
                pltpu.sync_copy(dst_hbm.at[pl.ds(base, CPS)], dst_v)
                for b in range(NBUF):
                    pltpu.async_copy(table_hbm.at[src_v.at[b]], rows[b],
                                     sg[b])

                def grp(q, carry2):
                    j0 = NBUF * q
                    for b in range(NBUF):
                        pltpu.make_async_copy(table_hbm.at[src_v.at[j0 + b]],
                                              rows[b], sg[b]).wait()
                        pltpu.async_copy(rows[b], acc.at[dst_v.at[j0 + b]],
                                         ss[b], add=True)
                    for b in range(NBUF):
                        jn = j0 + b + NBUF

                        @pl.when(jn < CPS)
                        def _():
                            pltpu.make_async_copy(rows[b],
                                                  acc.at[dst_v.at[b]],
                                                  ss[b]).wait()
                            pltpu.async_copy(table_hbm.at[src_v.at[jn]],
                                             rows[b], sg[b])
                    return carry2

                lax.fori_loop(0, CPS // NBUF, grp, 0)
                for b in range(NBUF):
                    pltpu.make_async_copy(rows[b], acc.at[dst_v.at[b]],
                                          ss[b]).wait()
                return carry

            lax.fori_loop(0, NSTAGE, stage, 0)

        zero_acc()
        pltpu.sync_copy(b1_hbm.at[c], b1v)
        plsc.subcore_barrier()
        segsum(g1_hbm)
        plsc.subcore_barrier()

        def relu_piece(p, carry):
            r0 = s * HPT + p * HPIECE
            buf = rows[0].at[pl.ds(0, HPIECE)]
            pltpu.sync_copy(acc.at[pl.ds(r0, HPIECE)], buf)

            def row(r, carry2):
                for k in range(4):
                    sl = pl.ds(16 * k, 16)
                    rows[0][r, sl] = jnp.maximum(rows[0][r, sl] + b1v[sl],
                                                 0.0)
                return carry2

            lax.fori_loop(0, HPIECE, row, 0)
            pltpu.sync_copy(buf, h1_hbm.at[pl.ds(c * N + r0, HPIECE)])
            return carry

        lax.fori_loop(0, N // NSUB // HPIECE, relu_piece, 0)
        plsc.subcore_barrier()

        zero_acc()
        plsc.subcore_barrier()
        segsum(h1_hbm)
        plsc.subcore_barrier()
        pltpu.sync_copy(acc.at[pl.ds(s * RPT, RPT)],
                        out_hbm.at[c, pl.ds(s * RPT, RPT)])

    return two_convs


_two_convs = _make_two_convs()


def _mm_split_body(x_ref, w_ref, o_ref):
    o_ref[...] = jnp.dot(x_ref[...], w_ref[0],
                         preferred_element_type=jnp.float32)


def _mm_split(x, w):
    return pl.pallas_call(
        _mm_split_body,
        grid=(GRID, 2),
        in_specs=[
            pl.BlockSpec((BN, D), lambda i, j: (i, 0)),
            pl.BlockSpec((1, D, HALF), lambda i, j: (j, 0, 0)),
        ],
        out_specs=pl.BlockSpec((BN, HALF), lambda i, j: (j * GRID + i, 0)),
        out_shape=jax.ShapeDtypeStruct((2 * N, HALF), jnp.float32),
    )(x, w)


def _final_body(p_ref, w2, b2, cond, wf, bf, wm, bm, wl, bl,
                zm_ref, zl_ref, acc_ref):
    i = pl.program_id(0)
    agg = jnp.concatenate([p_ref[0], p_ref[1]], axis=1)
    h = jnp.maximum(
        jnp.dot(agg, w2[...], preferred_element_type=jnp.float32)
        + b2[...], 0.0)
    bsum = jnp.sum(h, axis=0, keepdims=True)

    @pl.when(i == 0)
    def _():
        acc_ref[...] = bsum

    @pl.when(i > 0)
    def _():
        acc_ref[...] += bsum

    @pl.when(i == GRID - 1)
    def _():
        pooled = acc_ref[...] / N
        cat = jnp.concatenate([pooled, cond[...]], axis=1)
        z = jnp.maximum(
            jnp.dot(cat, wf[...], preferred_element_type=jnp.float32)
            + bf[...], 0.0)
        zm_ref[...] = jnp.dot(z, wm[...],
                              preferred_element_type=jnp.float32) + bm[...]
        zl_ref[...] = jnp.dot(z, wl[...],
                              preferred_element_type=jnp.float32) + bl[...]


def _final(p3, w2, b2, cond, wf, bf, wm, bm, wl, bl):
    full = lambda i: (0, 0)
    return pl.pallas_call(
        _final_body,
        grid=(GRID,),
        in_specs=[
            pl.BlockSpec((2, BN, HALF), lambda i: (0, i, 0)),
            pl.BlockSpec((HID, HID), full),
            pl.BlockSpec((1, HID), full),
            pl.BlockSpec((1, COND), full),
            pl.BlockSpec((HID + COND, LAT), full),
            pl.BlockSpec((1, LAT), full),
            pl.BlockSpec((LAT, LAT), full),
            pl.BlockSpec((1, LAT), full),
            pl.BlockSpec((LAT, LAT), full),
            pl.BlockSpec((1, LAT), full),
        ],
        out_specs=[pl.BlockSpec((1, LAT), full), pl.BlockSpec((1, LAT), full)],
        out_shape=[jax.ShapeDtypeStruct((1, LAT), jnp.float32),
                   jax.ShapeDtypeStruct((1, LAT), jnp.float32)],
        scratch_shapes=[pltpu.VMEM((1, HID), jnp.float32)],
    )(p3, w2, b2, cond, wf, bf, wm, bm, wl, bl)


def kernel(x, edge_index, conditions, W1, b1, W2, b2, Wf, bf, Wm, bm, Wl, bl):
    src = jnp.concatenate([edge_index[0], _PAD_SRC])
    dst = jnp.concatenate([edge_index[1], _PAD_DST])
    srcs = jnp.concatenate([src, N + src]).reshape(2 * NROWS, CHUNK)
    dst_r = dst.reshape(NROWS, CHUNK)
    zeros = jnp.zeros((N_PAD, HALF), jnp.float32)

    w1s = jnp.stack([W1[:, :HALF], W1[:, HALF:]])
    g1 = _mm_split(x, w1s)
    _, p2 = _two_convs(g1, b1.reshape(2, HALF), srcs, dst_r, zeros)
    z_mean, z_logvar = _final(p2, W2,
                              b2.reshape(1, HID), conditions,
                              Wf, bf.reshape(1, LAT), Wm, bm.reshape(1, LAT),
                              Wl, bl.reshape(1, LAT))
    return (z_mean, z_logvar)


# speedup vs baseline: 1.1712x; 1.1712x over previous
import functools

import jax
import jax.numpy as jnp
from jax import lax
from jax.experimental import pallas as pl
from jax.experimental.pallas import tpu as pltpu
from jax.experimental.pallas import tpu_sc as plsc

N = 10000
E = 320000
D = 128
HID = 128
LAT = 64
COND = 5

NCORES = 2
NSUB = 16
HALF = D // 2
CHUNK = 128
NROWS = NSUB * 160
CPT = NROWS // NSUB
NSTAGE = 2
CPS = CPT // NSTAGE
NBUF = 8
E_PAD = NROWS * CHUNK
N_PAD = 10112
RPT = N_PAD // NSUB

BN = 2000
GRID = N // BN


def _make_segsum():
    mesh = plsc.VectorSubcoreMesh(core_axis_name="c", subcore_axis_name="s",
                                  num_cores=NCORES, num_subcores=NSUB)

    @functools.partial(
        pl.kernel,
        out_type=jax.ShapeDtypeStruct((NCORES * N_PAD, HALF), jnp.float32),
        mesh=mesh,
        compiler_params=pltpu.CompilerParams(use_tc_tiling_on_sc=False),
        scratch_types=(
            [pltpu.VMEM((CPS, CHUNK), jnp.int32),
             pltpu.VMEM((CPS, CHUNK), jnp.int32)]
            + [pltpu.VMEM((CHUNK, HALF), jnp.float32)] * NBUF
            + [pltpu.VMEM_SHARED((N_PAD, HALF), jnp.float32)]
            + [pltpu.SemaphoreType.DMA] * (2 * NBUF)
        ),
    )
    def segsum(h_hbm, src_hbm, dst_hbm, zeros_hbm, out_hbm,
               src_v, dst_v, *rest):
        rows = list(rest[:NBUF])
        acc = rest[NBUF]
        sg = list(rest[NBUF + 1:NBUF + 1 + NBUF])
        ss = list(rest[NBUF + 1 + NBUF:])
        c = lax.axis_index("c")
        s = lax.axis_index("s")
        pltpu.sync_copy(zeros_hbm.at[pl.ds(s * RPT, RPT)],
                        acc.at[pl.ds(s * RPT, RPT)])
        plsc.subcore_barrier()

        def stage(st, carry):
            base = s * CPT + st * CPS
            pltpu.sync_copy(src_hbm.at[pl.ds(c * NROWS + base, CPS)], src_v)
            pltpu.sync_copy(dst_hbm.at[pl.ds(base, CPS)], dst_v)
            for b in range(NBUF):
                pltpu.async_copy(h_hbm.at[src_v.at[b]], rows[b], sg[b])

            def quad(q, carry2):
                j0 = NBUF * q
                for b in range(NBUF):
                    pltpu.make_async_copy(h_hbm.at[src_v.at[j0 + b]], rows[b],
                                          sg[b]).wait()
                    pltpu.async_copy(rows[b], acc.at[dst_v.at[j0 + b]], ss[b],
                                     add=True)
                for b in range(NBUF):
                    jn = j0 + b + NBUF

                    @pl.when(jn < CPS)
                    def _():
                        pltpu.make_async_copy(rows[b], acc.at[dst_v.at[b]],
                                              ss[b]).wait()
                        pltpu.async_copy(h_hbm.at[src_v.at[jn]], rows[b],
                                         sg[b])
                return carry2

            lax.fori_loop(0, CPS // NBUF, quad, 0)
            for b in range(NBUF):
                pltpu.make_async_copy(rows[b], acc.at[dst_v.at[b]],
                                      ss[b]).wait()
            return carry

        lax.fori_loop(0, NSTAGE, stage, 0)
        plsc.subcore_barrier()
        pltpu.sync_copy(acc.at[pl.ds(s * RPT, RPT)],
                        out_hbm.at[pl.ds(c * N_PAD + s * RPT, RPT)])

    return segsum


_segsum = _make_segsum()


def _mm_body(p_ref, w_ref, b_ref, o_ref):
    agg = jnp.concatenate([p_ref[0], p_ref[1]], axis=1)
    o_ref[...] = jnp.maximum(
        jnp.dot(agg, w_ref[...], preferred_element_type=jnp.float32)
        + b_ref[...], 0.0)


def _combine_mm(p3, w, b):
    return pl.pallas_call(
        _mm_body,
        grid=(GRID,),
        in_specs=[
            pl.BlockSpec((2, BN, HALF), lambda i: (0, i, 0)),
            pl.BlockSpec((D, HID), lambda i: (0, 0)),
            pl.BlockSpec((1, HID), lambda i: (0, 0)),
        ],
        out_specs=pl.BlockSpec((BN, HID), lambda i: (i, 0)),
        out_shape=jax.ShapeDtypeStruct((N, HID), jnp.float32),
    )(p3, w, b)


def _final_body(p_ref, w2, b2, cond, wf, bf, wm, bm, wl, bl,
                zm_ref, zl_ref, acc_ref):
    i = pl.program_id(0)
    agg = jnp.concatenate([p_ref[0], p_ref[1]], axis=1)
    h = jnp.maximum(
        jnp.dot(agg, w2[...], preferred_element_type=jnp.float32)
        + b2[...], 0.0)
    bsum = jnp.sum(h, axis=0, keepdims=True)

    @pl.when(i == 0)
    def _():
        acc_ref[...] = bsum

    @pl.when(i > 0)
    def _():
        acc_ref[...] += bsum

    @pl.when(i == GRID - 1)
    def _():
        pooled = acc_ref[...] / N
        cat = jnp.concatenate([pooled, cond[...]], axis=1)
        z = jnp.maximum(
            jnp.dot(cat, wf[...], preferred_element_type=jnp.float32)
            + bf[...], 0.0)
        zm_ref[...] = jnp.dot(z, wm[...],
                              preferred_element_type=jnp.float32) + bm[...]
        zl_ref[...] = jnp.dot(z, wl[...],
                              preferred_element_type=jnp.float32) + bl[...]


def _final(p3, w2, b2, cond, wf, bf, wm, bm, wl, bl):
    full = lambda i: (0, 0)
    return pl.pallas_call(
        _final_body,
        grid=(GRID,),
        in_specs=[
            pl.BlockSpec((2, BN, HALF), lambda i: (0, i, 0)),
            pl.BlockSpec((HID, HID), full),
            pl.BlockSpec((1, HID), full),
            pl.BlockSpec((1, COND), full),
            pl.BlockSpec((HID + COND, LAT), full),
            pl.BlockSpec((1, LAT), full),
            pl.BlockSpec((LAT, LAT), full),
            pl.BlockSpec((1, LAT), full),
            pl.BlockSpec((LAT, LAT), full),
            pl.BlockSpec((1, LAT), full),
        ],
        out_specs=[pl.BlockSpec((1, LAT), full), pl.BlockSpec((1, LAT), full)],
        out_shape=[jax.ShapeDtypeStruct((1, LAT), jnp.float32),
                   jax.ShapeDtypeStruct((1, LAT), jnp.float32)],
        scratch_shapes=[pltpu.VMEM((1, HID), jnp.float32)],
    )(p3, w2, b2, cond, wf, bf, wm, bm, wl, bl)


def kernel(x, edge_index, conditions, W1, b1, W2, b2, Wf, bf, Wm, bm, Wl, bl):
    npad = E_PAD - E
    pad_i = jnp.arange(npad, dtype=jnp.int32)
    src = jnp.concatenate([edge_index[0], pad_i % N])
    dst = jnp.concatenate([edge_index[1], N + pad_i % (N_PAD - N)])
    srcs = jnp.concatenate([2 * src, 2 * src + 1]).reshape(2 * NROWS, CHUNK)
    dst_r = dst.reshape(NROWS, CHUNK)
    zeros = jnp.zeros((N_PAD, HALF), jnp.float32)

    b1r = b1.reshape(1, HID)
    b2r = b2.reshape(1, HID)

    xr = x.reshape(2 * N, HALF)
    p1 = _segsum(xr, srcs, dst_r, zeros).reshape(2, N_PAD, HALF)
    h1 = _combine_mm(p1, W1, b1r)
    p2 = _segsum(h1.reshape(2 * N, HALF), srcs, dst_r,
                 zeros).reshape(2, N_PAD, HALF)
    z_mean, z_logvar = _final(p2, W2, b2r, conditions,
                              Wf, bf.reshape(1, LAT), Wm, bm.reshape(1, LAT),
                              Wl, bl.reshape(1, LAT))
    return (z_mean, z_logvar)
